# baseline (speedup 1.0000x reference)
import functools

import jax
import jax.numpy as jnp
from jax import lax
from jax.experimental import pallas as pl
from jax.experimental.pallas import tpu as pltpu
from jax.experimental.pallas import tpu_sc as plsc

_B = 4096
_S = 200
_D = 64
_NW = 32
_BB = _B // _NW
_DT = _D // 8
_EPS = 1e-08


def _dec_body(lens_ref, dec_ref):
    s0 = pl.program_id(0) * 8
    lens = lens_ref[...]
    for si in range(8):
        m = jnp.where(lens > (s0 + si), jnp.float32(1.0 + _EPS),
                      jnp.float32(_EPS))
        dec_ref[si] = jnp.broadcast_to(m[None, :, None, :],
                                       (_DT, _NW, 8, 128))


def _feeder_body(tokens_h, lens_h, emb_h, x_h, ts_h,
                 tk_v, len_v, lens_full_v, ts_v,
                 g0_v, g1_v, xt0_v, xt1_v,
                 col_v, dh_v, i1_v,
                 gsem, xsem):
    cid = lax.axis_index("c")
    sid = lax.axis_index("s")
    wid = cid * 16 + sid
    b0 = wid * _BB

    pltpu.sync_copy(tokens_h.at[:, wid], tk_v)
    pltpu.sync_copy(lens_h.at[pl.ds(b0, _BB)], len_v)

    @pl.when(wid == 0)
    def _():
        pltpu.sync_copy(lens_h, lens_full_v.at[pl.ds(0, _B)])

        def mx(i, acc):
            return jnp.maximum(acc, lens_full_v[pl.ds(i * 16, 16)])

        m = lax.fori_loop(0, _B // 16, mx, jnp.zeros((16,), jnp.int32))
        m = jnp.minimum(m, _S)
        mm = m[0]
        for j in range(1, 16):
            mm = jnp.maximum(mm, m[j])
        ts_v[:] = jnp.full((16,), mm, jnp.int32)
        pltpu.sync_copy(ts_v, ts_h)

    iota = lax.iota(jnp.int32, 16)
    lens_k = [len_v[pl.ds(k * 16, 16)] for k in range(8)]

    def tbl_body(j, carry):
        a = jnp.bitwise_and(iota + j, 15)
        for kd in range(4):
            jk = j * 4 + kd
            d = a + kd * 16
            col_v[jk, pl.ds(0, 16)] = d
            dh_v[jk, pl.ds(0, 16)] = jnp.right_shift(d, 3)
            i1_v[jk, pl.ds(0, 16)] = (
                jnp.left_shift(jnp.bitwise_and(d, 7), 7) + iota)
        return carry

    lax.fori_loop(0, 16, tbl_body, 0)

    def issue_gather(s, g_v):
        pltpu.async_copy(emb_h.at[tk_v.at[s // 8, s % 8]], g_v, gsem)

    def wait_gather(s, g_v):
        pltpu.make_async_copy(emb_h.at[tk_v.at[s // 8, s % 8]], g_v,
                              gsem).wait()

    def process(s, g_v, xt_v):
        ms = [jnp.where(lens_k[k] > s, jnp.float32(1.0), jnp.float32(0.0))
              for k in range(8)]

        @plsc.parallel_loop(0, 64, 1, unroll=2, carry=tuple(ms))
        def jk_body(jk, carry):
            col = col_v[jk, pl.ds(0, 16)]
            dh = dh_v[jk, pl.ds(0, 16)]
            i1 = i1_v[jk, pl.ds(0, 16)]
            for kb in range(8):
                row = iota + (kb * 16)
                v = plsc.load_gather(g_v, [row, col])
                inner = i1 + (kb * 16)
                plsc.store_scatter(xt_v, [dh, inner], v * carry[kb])
            return carry

        for dh in range(_DT):
            pltpu.async_copy(xt_v.at[dh], x_h.at[s, dh, wid], xsem)

    def drain_outputs(s):
        for dh in range(_DT):
            for _ in range(2):
                pltpu.make_async_copy(xt0_v.at[dh], x_h.at[s, dh, wid],
                                      xsem).wait()

    issue_gather(0, g0_v)

    def pair_body(j, carry):
        s0 = 2 * j
        s1 = s0 + 1
        issue_gather(s1, g1_v)

        @pl.when(j > 0)
        def _():
            drain_outputs(s0)

        wait_gather(s0, g0_v)
        process(s0, g0_v, xt0_v)

        @pl.when(j < _S // 2 - 1)
        def _():
            issue_gather(s0 + 2, g0_v)

        wait_gather(s1, g1_v)
        process(s1, g1_v, xt1_v)
        return carry

    lax.fori_loop(0, _S // 2, pair_body, 0)
    drain_outputs(0)


def kernel(tokens, row_lengths, max_sequence_length, embeddings):
    del max_sequence_length
    tk = tokens.T.reshape(_S // 8, 8, _NW, _BB).transpose(0, 2, 1, 3)

    feeder = pl.kernel(
        _feeder_body,
        out_type=(
            jax.ShapeDtypeStruct((_S, _DT, _NW, 1024), jnp.float32),
            jax.ShapeDtypeStruct((16,), jnp.int32),
        ),
        mesh=plsc.VectorSubcoreMesh(core_axis_name="c", subcore_axis_name="s"),
        compiler_params=pltpu.CompilerParams(use_tc_tiling_on_sc=False,
                                             needs_layout_passes=False),
        scratch_types=[
            pltpu.VMEM((_S // 8, 8, _BB), jnp.int32),
            pltpu.VMEM((_BB,), jnp.int32),
            pltpu.VMEM((_B + 16,), jnp.int32),
            pltpu.VMEM((16,), jnp.int32),
            pltpu.VMEM((_BB, _D), jnp.float32),
            pltpu.VMEM((_BB, _D), jnp.float32),
            pltpu.VMEM((_DT, 1024), jnp.float32),
            pltpu.VMEM((_DT, 1024), jnp.float32),
            pltpu.VMEM((64, 16), jnp.int32),
            pltpu.VMEM((64, 16), jnp.int32),
            pltpu.VMEM((64, 16), jnp.int32),
            pltpu.SemaphoreType.DMA,
            pltpu.SemaphoreType.DMA,
        ],
    )
    xo, ts = feeder(tk, row_lengths, embeddings)

    dec5 = pl.pallas_call(
        _dec_body,
        grid=(_S // 8,),
        in_specs=[pl.BlockSpec((_NW, 128), lambda s: (0, 0))],
        out_specs=pl.BlockSpec((8, _DT, _NW, 8, 128),
                               lambda s: (s, 0, 0, 0, 0)),
        out_shape=jax.ShapeDtypeStruct((_S, _DT, _NW, 8, 128), jnp.float32),
    )(row_lengths.reshape(_NW, 128))

    xo = xo.reshape(_S, _DT, _NW, 8, 128)
    x = xo.transpose(2, 4, 0, 1, 3).reshape(_B, _S, _D)
    dec = dec5.transpose(2, 4, 0, 1, 3).reshape(_B, _S, _D)
    return (x, dec, ts[0])


# speedup vs baseline: 1.0202x; 1.0202x over previous
import functools

import jax
import jax.numpy as jnp
from jax import lax
from jax.experimental import pallas as pl
from jax.experimental.pallas import tpu as pltpu
from jax.experimental.pallas import tpu_sc as plsc

_B = 4096
_S = 200
_D = 64
_NW = 32
_BB = _B // _NW
_DT = _D // 8
_EPS = 1e-08


def _dec_body(lens_ref, dec_ref):
    s = pl.program_id(0)
    m = jnp.where(lens_ref[...] > s, jnp.float32(1.0 + _EPS),
                  jnp.float32(_EPS))
    dec_ref[...] = jnp.broadcast_to(m[None, None, :, None, :],
                                    (1, _DT, _NW, 8, 128))


def _feeder_body(tokens_h, lens_h, emb_h, x_h, ts_h,
                 tk_v, len_v, lens_full_v, ts_v,
                 g0_v, g1_v, xt0_v, xt1_v,
                 col_v, dh_v, i1_v,
                 gsem, xsem):
    cid = lax.axis_index("c")
    sid = lax.axis_index("s")
    wid = cid * 16 + sid
    b0 = wid * _BB

    pltpu.sync_copy(tokens_h.at[:, pl.ds(b0, _BB)], tk_v)
    pltpu.sync_copy(lens_h.at[pl.ds(b0, _BB)], len_v)

    @pl.when(wid == 0)
    def _():
        pltpu.sync_copy(lens_h, lens_full_v.at[pl.ds(0, _B)])

        def mx(i, acc):
            return jnp.maximum(acc, lens_full_v[pl.ds(i * 16, 16)])

        m = lax.fori_loop(0, _B // 16, mx, jnp.zeros((16,), jnp.int32))
        m = jnp.minimum(m, _S)
        mm = m[0]
        for j in range(1, 16):
            mm = jnp.maximum(mm, m[j])
        ts_v[:] = jnp.full((16,), mm, jnp.int32)
        pltpu.sync_copy(ts_v, ts_h)

    iota = lax.iota(jnp.int32, 16)
    lens_k = [len_v[pl.ds(k * 16, 16)] for k in range(8)]

    def tbl_body(j, carry):
        a = jnp.bitwise_and(iota + j, 15)
        for kd in range(4):
            jk = j * 4 + kd
            d = a + kd * 16
            col_v[jk, pl.ds(0, 16)] = d
            dh_v[jk, pl.ds(0, 16)] = jnp.right_shift(d, 3)
            i1_v[jk, pl.ds(0, 16)] = (
                jnp.left_shift(jnp.bitwise_and(d, 7), 7) + iota)
        return carry

    lax.fori_loop(0, 16, tbl_body, 0)

    def issue_gather(s, g_v):
        pltpu.async_copy(emb_h.at[tk_v.at[s]], g_v, gsem)

    def wait_gather(s, g_v):
        pltpu.make_async_copy(emb_h.at[tk_v.at[s]], g_v, gsem).wait()

    def process(s, g_v, xt_v):
        ms = [jnp.where(lens_k[k] > s, jnp.float32(1.0), jnp.float32(0.0))
              for k in range(8)]

        @plsc.parallel_loop(0, 64, 1, unroll=2, carry=tuple(ms))
        def jk_body(jk, carry):
            col = col_v[jk, pl.ds(0, 16)]
            dh = dh_v[jk, pl.ds(0, 16)]
            i1 = i1_v[jk, pl.ds(0, 16)]
            for kb in range(8):
                row = iota + (kb * 16)
                v = plsc.load_gather(g_v, [row, col])
                inner = i1 + (kb * 16)
                plsc.store_scatter(xt_v, [dh, inner], v * carry[kb])
            return carry

        for dh in range(_DT):
            pltpu.async_copy(xt_v.at[dh], x_h.at[s, dh, wid], xsem)

    def drain_outputs(s):
        for dh in range(_DT):
            for _ in range(2):
                pltpu.make_async_copy(xt0_v.at[dh], x_h.at[s, dh, wid],
                                      xsem).wait()

    issue_gather(0, g0_v)

    def pair_body(j, carry):
        s0 = 2 * j
        s1 = s0 + 1
        issue_gather(s1, g1_v)

        @pl.when(j > 0)
        def _():
            drain_outputs(s0)

        wait_gather(s0, g0_v)
        process(s0, g0_v, xt0_v)

        @pl.when(j < _S // 2 - 1)
        def _():
            issue_gather(s0 + 2, g0_v)

        wait_gather(s1, g1_v)
        process(s1, g1_v, xt1_v)
        return carry

    lax.fori_loop(0, _S // 2, pair_body, 0)
    drain_outputs(0)


def kernel(tokens, row_lengths, max_sequence_length, embeddings):
    del max_sequence_length
    tk = tokens.T

    feeder = pl.kernel(
        _feeder_body,
        out_type=(
            jax.ShapeDtypeStruct((_S, _DT, _NW, 1024), jnp.float32),
            jax.ShapeDtypeStruct((16,), jnp.int32),
        ),
        mesh=plsc.VectorSubcoreMesh(core_axis_name="c", subcore_axis_name="s"),
        compiler_params=pltpu.CompilerParams(use_tc_tiling_on_sc=False,
                                             needs_layout_passes=False),
        scratch_types=[
            pltpu.VMEM((_S, _BB), jnp.int32),
            pltpu.VMEM((_BB,), jnp.int32),
            pltpu.VMEM((_B + 16,), jnp.int32),
            pltpu.VMEM((16,), jnp.int32),
            pltpu.VMEM((_BB, _D), jnp.float32),
            pltpu.VMEM((_BB, _D), jnp.float32),
            pltpu.VMEM((_DT, 1024), jnp.float32),
            pltpu.VMEM((_DT, 1024), jnp.float32),
            pltpu.VMEM((64, 16), jnp.int32),
            pltpu.VMEM((64, 16), jnp.int32),
            pltpu.VMEM((64, 16), jnp.int32),
            pltpu.SemaphoreType.DMA,
            pltpu.SemaphoreType.DMA,
        ],
    )
    xo, ts = feeder(tk, row_lengths, embeddings)

    dec5 = pl.pallas_call(
        _dec_body,
        grid=(_S,),
        in_specs=[pl.BlockSpec((_NW, 128), lambda s: (0, 0))],
        out_specs=pl.BlockSpec((1, _DT, _NW, 8, 128),
                               lambda s: (s, 0, 0, 0, 0)),
        out_shape=jax.ShapeDtypeStruct((_S, _DT, _NW, 8, 128), jnp.float32),
    )(row_lengths.reshape(_NW, 128))

    xo = xo.reshape(_S, _DT, _NW, 8, 128)
    x = xo.transpose(2, 4, 0, 1, 3).reshape(_B, _S, _D)
    dec = dec5.transpose(2, 4, 0, 1, 3).reshape(_B, _S, _D)
    return (x, dec, ts[0])
